# baseline (speedup 1.0000x reference)
import functools

import jax
import jax.numpy as jnp
from jax import lax
from jax.experimental import pallas as pl
from jax.experimental.pallas import tpu as pltpu
from jax.experimental.pallas import tpu_sc as plsc

C, H, W = 1024, 160, 160
THRESHOLD = 0.2
PATCH = 14.0
NMS_RADIUS = 4
MAX_KEYPOINTS = 256

HW = H * W
LANES = 16
NWORK = 32
NTILE = HW // 128
SPW = 7
CC = 256
NCHUNK = C // CC
NSTEP = SPW * NCHUNK
NEG_FILL = -1e9


def _sc_reduce_body(feat_hbm, out_hbm, buf0, buf1, stage, sem0, sem1):
    wid = lax.axis_index("s") * 2 + lax.axis_index("c")

    bufs = (buf0, buf1)
    sems = (sem0, sem1)

    def stripe_col(t):
        s = jnp.minimum((t // NCHUNK) * NWORK + wid, NTILE - 1)
        return s * 128

    def start(t):
        col = pl.multiple_of(stripe_col(t), 128)
        pltpu.make_async_copy(
            feat_hbm.at[pl.ds((t % NCHUNK) * CC, CC), pl.ds(col, 128)],
            bufs[t % 2], sems[t % 2]).start()

    def wait(t):
        pltpu.make_async_copy(
            feat_hbm.at[pl.ds(0, CC), pl.ds(0, 128)],
            bufs[t % 2], sems[t % 2]).wait()

    start(0)
    start(1)

    for i in range(SPW):
        accs = [jnp.zeros((LANES,), jnp.float32) for _ in range(8)]
        for cchunk in range(NCHUNK):
            t = i * NCHUNK + cchunk
            wait(t)
            buf = bufs[t % 2]

            def body(r, carry):
                out = []
                for j in range(8):
                    v = buf[r, pl.ds(j * LANES, LANES)]
                    out.append(carry[j] + v * v)
                return tuple(out)

            accs = lax.fori_loop(0, CC, body, tuple(accs), unroll=2)
            if t + 2 < NSTEP:
                start(t + 2)
        for j in range(8):
            stage[pl.ds(j * LANES, LANES)] = accs[j]
        col = pl.multiple_of(stripe_col(i * NCHUNK), 128)
        pltpu.sync_copy(stage, out_hbm.at[pl.ds(col, 128)])


@functools.partial(
    pl.kernel,
    out_type=jax.ShapeDtypeStruct((HW,), jnp.float32),
    mesh=plsc.VectorSubcoreMesh(core_axis_name="c", subcore_axis_name="s"),
    scratch_types=[
        pltpu.VMEM((CC, 128), jnp.float32),
        pltpu.VMEM((CC, 128), jnp.float32),
        pltpu.VMEM((128,), jnp.float32),
        pltpu.SemaphoreType.DMA,
        pltpu.SemaphoreType.DMA,
    ],
)
def _sc_reduce(feat_hbm, out_hbm, buf0, buf1, stage, sem0, sem1):
    _sc_reduce_body(feat_hbm, out_hbm, buf0, buf1, stage, sem0, sem1)


def _tc_tail_body(acc_ref, xy_ref, scores_ref):
    resp = jnp.sqrt(acc_ref[...])

    ninf = jnp.full((H, NMS_RADIUS), -jnp.inf, jnp.float32)
    padded = jnp.concatenate([ninf, resp, ninf], axis=1)
    hp = padded[:, 0:W]
    for s in range(1, 2 * NMS_RADIUS + 1):
        hp = jnp.maximum(hp, padded[:, s:s + W])

    ninf2 = jnp.full((NMS_RADIUS, W), -jnp.inf, jnp.float32)
    padded2 = jnp.concatenate([ninf2, hp, ninf2], axis=0)
    pooled = padded2[0:H, :]
    for s in range(1, 2 * NMS_RADIUS + 1):
        pooled = jnp.maximum(pooled, padded2[s:s + H, :])

    keep = (resp > THRESHOLD) & (resp == pooled)
    m = jnp.where(keep, resp, NEG_FILL)

    row_iota = lax.broadcasted_iota(jnp.int32, (H, W), 0)
    col_iota = lax.broadcasted_iota(jnp.int32, (H, W), 1)
    fidx = (row_iota * W + col_iota).astype(jnp.float32)

    mv = m.reshape(H // 5, 5, W)
    fv = fidx.reshape(H // 5, 5, W)
    vals, idxs = mv[:, 0], fv[:, 0]
    for dr in range(1, 5):
        v2, i2 = mv[:, dr], fv[:, dr]
        take = v2 > vals
        vals = jnp.where(take, v2, vals)
        idxs = jnp.where(take, i2, idxs)
    tv = vals.T.reshape(W // 5, 5, H // 5)
    ti = idxs.T.reshape(W // 5, 5, H // 5)
    bvals, bidx = tv[:, 0], ti[:, 0]
    for dc in range(1, 5):
        v2, i2 = tv[:, dc], ti[:, dc]
        take = (v2 > bvals) | ((v2 == bvals) & (i2 < bidx))
        bvals = jnp.where(take, v2, bvals)
        bidx = jnp.where(take, i2, bidx)

    nblk = H // 5
    vj = jnp.concatenate([bvals[r:r + 1, :] for r in range(nblk)], axis=1)
    ij = jnp.concatenate([bidx[r:r + 1, :] for r in range(nblk)], axis=1)
    vi = jnp.concatenate([bvals[:, c:c + 1] for c in range(nblk)], axis=0)
    ii = jnp.concatenate([bidx[:, c:c + 1] for c in range(nblk)], axis=0)
    beats = (vj > vi) | ((vj == vi) & (ij < ii))
    beats2 = (~beats) & (ij != ii)
    rank_col = jnp.sum(beats.astype(jnp.float32), axis=1, keepdims=True)
    rank_row = jnp.sum(beats2.astype(jnp.float32), axis=0, keepdims=True)

    p_col = lax.broadcasted_iota(
        jnp.int32, (MAX_KEYPOINTS, 1), 0).astype(jnp.float32)
    p_row = lax.broadcasted_iota(
        jnp.int32, (1, MAX_KEYPOINTS), 1).astype(jnp.float32)
    onehot_a = (rank_row == p_col).astype(jnp.float32)
    idxsel = jnp.sum(onehot_a * ij, axis=1, keepdims=True)
    onehot_b = (rank_col == p_row).astype(jnp.float32)
    scores = jnp.sum(onehot_b * vi, axis=0)

    idx_i = idxsel.astype(jnp.int32)
    r_out = (idx_i // W).astype(jnp.float32)
    c_out = (idx_i % W).astype(jnp.float32)
    scores_ref[...] = scores
    xy_ref[...] = jnp.concatenate([c_out * PATCH, r_out * PATCH], axis=1)


def kernel(feat_map, nms_radius, max_keypoints):
    del nms_radius, max_keypoints
    feat2 = feat_map.reshape(C, HW)
    sumsq = _sc_reduce(feat2)
    resp2d = sumsq.reshape(H, W)
    xy, scores = pl.pallas_call(
        _tc_tail_body,
        out_shape=[
            jax.ShapeDtypeStruct((MAX_KEYPOINTS, 2), jnp.float32),
            jax.ShapeDtypeStruct((MAX_KEYPOINTS,), jnp.float32),
        ],
    )(resp2d)
    return xy, scores


# speedup vs baseline: 1.2769x; 1.2769x over previous
import jax
import jax.numpy as jnp
from jax import lax
from jax.experimental import pallas as pl
from jax.experimental.pallas import tpu as pltpu

C, H, W = 1024, 160, 160
THRESHOLD = 0.2
PATCH = 14.0
NMS_RADIUS = 4
MAX_KEYPOINTS = 256

HW = H * W
CHUNK = 32
NBUF = 4
NCHUNK = C // CHUNK
NEG_FILL = -1e9


def _reduce_body(feat_hbm, out_ref, *scratch):
    bufs = scratch[:NBUF]
    sems = scratch[NBUF:]

    def start(c, b, tok=None):
        off = c * CHUNK if tok is None else pl.multiple_of(c * CHUNK + tok, 8)
        pltpu.make_async_copy(
            feat_hbm.at[pl.ds(off, CHUNK)], bufs[b], sems[b]).start()

    def wait(b):
        pltpu.make_async_copy(
            feat_hbm.at[pl.ds(0, CHUNK)], bufs[b], sems[b]).wait()

    for b in range(NBUF):
        start(b, b)

    acc = jnp.zeros((1, HW), jnp.float32)
    for t in range(NCHUNK):
        b = t % NBUF
        wait(b)
        x = bufs[b][...]
        acc = acc + jnp.sum(x * x, axis=0, keepdims=True)
        if t + NBUF < NCHUNK:
            tok = lax.shift_right_arithmetic(
                lax.bitcast_convert_type(jnp.max(acc), jnp.int32), 31)
            start(t + NBUF, b, tok)

    out_ref[...] = acc


def _tail_body(acc_ref, xy_ref, scores_ref):
    resp = jnp.sqrt(acc_ref[...])

    ninf = jnp.full((H, NMS_RADIUS), -jnp.inf, jnp.float32)
    padded = jnp.concatenate([ninf, resp, ninf], axis=1)
    hp = padded[:, 0:W]
    for s in range(1, 2 * NMS_RADIUS + 1):
        hp = jnp.maximum(hp, padded[:, s:s + W])

    ninf2 = jnp.full((NMS_RADIUS, W), -jnp.inf, jnp.float32)
    padded2 = jnp.concatenate([ninf2, hp, ninf2], axis=0)
    pooled = padded2[0:H, :]
    for s in range(1, 2 * NMS_RADIUS + 1):
        pooled = jnp.maximum(pooled, padded2[s:s + H, :])

    keep = (resp > THRESHOLD) & (resp == pooled)
    m = jnp.where(keep, resp, NEG_FILL)

    row_iota = lax.broadcasted_iota(jnp.int32, (H, W), 0)
    col_iota = lax.broadcasted_iota(jnp.int32, (H, W), 1)
    fidx = (row_iota * W + col_iota).astype(jnp.float32)

    mv = m.reshape(H // 5, 5, W)
    fv = fidx.reshape(H // 5, 5, W)
    vals, idxs = mv[:, 0], fv[:, 0]
    for dr in range(1, 5):
        v2, i2 = mv[:, dr], fv[:, dr]
        take = v2 > vals
        vals = jnp.where(take, v2, vals)
        idxs = jnp.where(take, i2, idxs)
    tv = vals.T.reshape(W // 5, 5, H // 5)
    ti = idxs.T.reshape(W // 5, 5, H // 5)
    bvals, bidx = tv[:, 0], ti[:, 0]
    for dc in range(1, 5):
        v2, i2 = tv[:, dc], ti[:, dc]
        take = (v2 > bvals) | ((v2 == bvals) & (i2 < bidx))
        bvals = jnp.where(take, v2, bvals)
        bidx = jnp.where(take, i2, bidx)

    nblk = H // 5
    vj = jnp.concatenate([bvals[r:r + 1, :] for r in range(nblk)], axis=1)
    ij = jnp.concatenate([bidx[r:r + 1, :] for r in range(nblk)], axis=1)
    vi = jnp.concatenate([bvals[:, c:c + 1] for c in range(nblk)], axis=0)
    ii = jnp.concatenate([bidx[:, c:c + 1] for c in range(nblk)], axis=0)
    beats = (vj > vi) | ((vj == vi) & (ij < ii))
    beats2 = (~beats) & (ij != ii)
    rank_col = jnp.sum(beats.astype(jnp.float32), axis=1, keepdims=True)
    rank_row = jnp.sum(beats2.astype(jnp.float32), axis=0, keepdims=True)

    p_col = lax.broadcasted_iota(
        jnp.int32, (MAX_KEYPOINTS, 1), 0).astype(jnp.float32)
    p_row = lax.broadcasted_iota(
        jnp.int32, (1, MAX_KEYPOINTS), 1).astype(jnp.float32)
    onehot_a = (rank_row == p_col).astype(jnp.float32)
    idxsel = jnp.sum(onehot_a * ij, axis=1, keepdims=True)
    onehot_b = (rank_col == p_row).astype(jnp.float32)
    scores = jnp.sum(onehot_b * vi, axis=0)

    idx_i = idxsel.astype(jnp.int32)
    r_out = (idx_i // W).astype(jnp.float32)
    c_out = (idx_i % W).astype(jnp.float32)
    scores_ref[...] = scores
    xy_ref[...] = jnp.concatenate([c_out * PATCH, r_out * PATCH], axis=1)


def kernel(feat_map, nms_radius, max_keypoints):
    del nms_radius, max_keypoints
    feat2 = feat_map.reshape(C, HW)
    sumsq = pl.pallas_call(
        _reduce_body,
        in_specs=[pl.BlockSpec(memory_space=pl.ANY)],
        out_specs=pl.BlockSpec(memory_space=pltpu.VMEM),
        out_shape=jax.ShapeDtypeStruct((1, HW), jnp.float32),
        scratch_shapes=(
            [pltpu.VMEM((CHUNK, HW), jnp.float32) for _ in range(NBUF)]
            + [pltpu.SemaphoreType.DMA for _ in range(NBUF)]
        ),
    )(feat2)
    resp2d = sumsq.reshape(H, W)
    xy, scores = pl.pallas_call(
        _tail_body,
        out_shape=[
            jax.ShapeDtypeStruct((MAX_KEYPOINTS, 2), jnp.float32),
            jax.ShapeDtypeStruct((MAX_KEYPOINTS,), jnp.float32),
        ],
    )(resp2d)
    return xy, scores
